# baseline (speedup 1.0000x reference)
import jax
import jax.numpy as jnp
from jax.experimental import pallas as pl


def _add_kernel(x_ref, t_ref, o_ref):
    o_ref[...] = x_ref[...] + t_ref[...][:, None, :]


def kernel(x, table):
    B, T, N, C = x.shape
    LANES = 128
    reps = LANES // C
    SUB = (N * C) // LANES
    table2 = jnp.concatenate([table] * reps, axis=1)
    xr = x.reshape(B * T, SUB, LANES)
    RB = 16
    grid = ((B * T) // RB,)
    out = pl.pallas_call(
        _add_kernel,
        grid=grid,
        in_specs=[
            pl.BlockSpec((RB, SUB, LANES), lambda i: (i, 0, 0)),
            pl.BlockSpec((RB, LANES), lambda i: (i % (T // RB), 0)),
        ],
        out_specs=pl.BlockSpec((RB, SUB, LANES), lambda i: (i, 0, 0)),
        out_shape=jax.ShapeDtypeStruct(xr.shape, x.dtype),
    )(xr, table2)
    return out.reshape(B, T, N, C)


# speedup vs baseline: 8.0104x; 8.0104x over previous
import jax
import jax.numpy as jnp
from jax.experimental import pallas as pl


def _add_kernel(x_ref, t_ref, o_ref):
    e = t_ref[0]
    o_ref[...] = x_ref[...] + e[:, :, None]


def kernel(x, table):
    B, T, N, C = x.shape
    xp = jnp.transpose(x, (0, 1, 3, 2)).reshape(B * T, C, N)
    RB = 16
    tblk = table[:T].reshape(T // RB, RB, C)
    grid = ((B * T) // RB,)
    out = pl.pallas_call(
        _add_kernel,
        grid=grid,
        in_specs=[
            pl.BlockSpec((RB, C, N), lambda i: (i, 0, 0)),
            pl.BlockSpec((1, RB, C), lambda i: (i % (T // RB), 0, 0)),
        ],
        out_specs=pl.BlockSpec((RB, C, N), lambda i: (i, 0, 0)),
        out_shape=jax.ShapeDtypeStruct(xp.shape, x.dtype),
    )(xp, tblk)
    return jnp.transpose(out.reshape(B, T, C, N), (0, 1, 3, 2))
